# baseline (speedup 1.0000x reference)
import functools

import jax
import jax.numpy as jnp
from jax import lax
from jax.experimental import pallas as pl
from jax.experimental.pallas import tpu as pltpu
from jax.experimental.pallas import tpu_sc as plsc

N_ROWS = 10000
N_PAD = 10240
N_INC = 320000
CH_F = 128
NC = 2
NS = 16
NW = NC * NS
CHUNK = 128
N_CH = 80
PER_W = N_CH * CHUNK
INC_PAD = NW * PER_W
FROWS = N_PAD // NS
ZBLK = 128

_mesh = plsc.VectorSubcoreMesh(core_axis_name="c", subcore_axis_name="s")


@functools.partial(
    pl.kernel,
    out_type=[
        jax.ShapeDtypeStruct((NC, N_PAD, CH_F), jnp.float32),
        jax.ShapeDtypeStruct((NC, N_PAD), jnp.float32),
    ],
    mesh=_mesh,
    scratch_types=[
        pltpu.VMEM((CHUNK,), jnp.int32),
        pltpu.VMEM((CHUNK,), jnp.int32),
        pltpu.VMEM((CHUNK, CH_F), jnp.float32),
        pltpu.VMEM((CHUNK,), jnp.float32),
        pltpu.VMEM((ZBLK, CH_F), jnp.float32),
        pltpu.VMEM((FROWS,), jnp.float32),
        pltpu.VMEM_SHARED((N_PAD, CH_F), jnp.float32),
        pltpu.VMEM_SHARED((N_PAD,), jnp.float32),
        pltpu.SemaphoreType.DMA,
    ],
)
def _sc_phase(table, gidx, sidx, acc_out, deg_out,
              gv, sv, rows, ones_v, zrow, zdeg,
              acc_sh, deg_sh, sem):
    c = lax.axis_index("c")
    s = lax.axis_index("s")
    wid = c * NS + s
    base = wid * PER_W

    zero16 = jnp.zeros((16,), jnp.float32)

    @pl.loop(0, CHUNK // 16)
    def _fill_ones(i):
        ones_v[pl.ds(i * 16, 16)] = jnp.ones((16,), jnp.float32)

    @pl.loop(0, ZBLK)
    def _fill_zrow(i):
        for j in range(CH_F // 16):
            zrow[i, pl.ds(j * 16, 16)] = zero16

    @pl.loop(0, FROWS // 16)
    def _fill_zdeg(i):
        zdeg[pl.ds(i * 16, 16)] = zero16

    z0 = s * FROWS
    for k in range(FROWS // ZBLK):
        pltpu.sync_copy(zrow, acc_sh.at[pl.ds(z0 + k * ZBLK, ZBLK)])
    pltpu.sync_copy(zdeg, deg_sh.at[pl.ds(z0, FROWS)])
    plsc.subcore_barrier()

    @pl.loop(0, N_CH)
    def _chunk(ci):
        off = base + ci * CHUNK
        pltpu.sync_copy(gidx.at[pl.ds(off, CHUNK)], gv)
        pltpu.sync_copy(sidx.at[pl.ds(off, CHUNK)], sv)
        pltpu.async_copy(table.at[gv], rows, sem).wait()
        pltpu.sync_copy(rows, acc_sh.at[sv], add=True)
        pltpu.sync_copy(ones_v, deg_sh.at[sv], add=True)

    plsc.subcore_barrier()

    pltpu.sync_copy(acc_sh.at[pl.ds(z0, FROWS)],
                    acc_out.at[c, pl.ds(z0, FROWS)])
    pltpu.sync_copy(deg_sh.at[pl.ds(z0, FROWS)],
                    deg_out.at[c, pl.ds(z0, FROWS)])


_RB = 1024


def _combine1_body(acc_ref, deg_ref, w_ref, out_ref):
    e = acc_ref[0] + acc_ref[1]
    d = (deg_ref[0] + deg_ref[1])[:, None]
    t = lax.dot_general(e, w_ref[...], (((1,), (1,)), ((), ())),
                        preferred_element_type=jnp.float32)
    out_ref[...] = jnp.where(d > 0, t / jnp.where(d > 0, d, 1.0), 0.0)


def _combine2_body(acc_ref, deg_ref, bias_ref, out_ref):
    e = acc_ref[0] + acc_ref[1]
    d = (deg_ref[0] + deg_ref[1])[:, None]
    out_ref[...] = jnp.where(d > 0, e / jnp.where(d > 0, d, 1.0), 0.0) \
        + bias_ref[...]


def _combine1(acc, deg, W):
    return pl.pallas_call(
        _combine1_body,
        grid=(N_PAD // _RB,),
        in_specs=[
            pl.BlockSpec((NC, _RB, CH_F), lambda i: (0, i, 0)),
            pl.BlockSpec((NC, _RB), lambda i: (0, i)),
            pl.BlockSpec((CH_F, CH_F), lambda i: (0, 0)),
        ],
        out_specs=pl.BlockSpec((_RB, CH_F), lambda i: (i, 0)),
        out_shape=jax.ShapeDtypeStruct((N_PAD, CH_F), jnp.float32),
    )(acc, deg, W)


def _combine2(acc, deg, bias2d):
    return pl.pallas_call(
        _combine2_body,
        grid=(N_PAD // _RB,),
        in_specs=[
            pl.BlockSpec((NC, _RB, CH_F), lambda i: (0, i, 0)),
            pl.BlockSpec((NC, _RB), lambda i: (0, i)),
            pl.BlockSpec((1, CH_F), lambda i: (0, 0)),
        ],
        out_specs=pl.BlockSpec((_RB, CH_F), lambda i: (i, 0)),
        out_shape=jax.ShapeDtypeStruct((N_PAD, CH_F), jnp.float32),
    )(acc, deg, bias2d)


def _pad1(idx, fill):
    pad = jnp.full((INC_PAD - N_INC,), fill, jnp.int32)
    return jnp.concatenate([idx, pad])


def kernel(x, hyperedge_index, W, bias):
    idx_node = hyperedge_index[0].astype(jnp.int32)
    idx_edge = hyperedge_index[1].astype(jnp.int32)
    node_g = _pad1(idx_node, 0)
    node_s = _pad1(idx_node, N_PAD - 1)
    edge_g = _pad1(idx_edge, 0)
    edge_s = _pad1(idx_edge, N_PAD - 1)

    acc_e, deg_e = _sc_phase(x, node_g, edge_s)
    t_edge = _combine1(acc_e, deg_e, W)
    acc_n, deg_n = _sc_phase(t_edge, edge_g, node_s)
    out = _combine2(acc_n, deg_n, bias.reshape(1, CH_F))
    return out[:N_ROWS]


# speedup vs baseline: 1.0002x; 1.0002x over previous
import functools

import jax
import jax.numpy as jnp
from jax import lax
from jax.experimental import pallas as pl
from jax.experimental.pallas import tpu as pltpu
from jax.experimental.pallas import tpu_sc as plsc

N_ROWS = 10000
N_PAD = 10240
N_INC = 320000
CH_F = 128
NC = 2
NS = 16
NW = NC * NS
CHUNK = 128
N_CH = 80
PER_W = N_CH * CHUNK
INC_PAD = NW * PER_W
FROWS = N_PAD // NS
ZBLK = 128

_mesh = plsc.VectorSubcoreMesh(core_axis_name="c", subcore_axis_name="s")


@functools.partial(
    pl.kernel,
    out_type=[
        jax.ShapeDtypeStruct((NC, N_PAD, CH_F), jnp.float32),
        jax.ShapeDtypeStruct((NC, N_PAD), jnp.float32),
    ],
    mesh=_mesh,
    scratch_types=[
        pltpu.VMEM((CHUNK,), jnp.int32),
        pltpu.VMEM((CHUNK,), jnp.int32),
        pltpu.VMEM((CHUNK, CH_F), jnp.float32),
        pltpu.VMEM((CHUNK,), jnp.float32),
        pltpu.VMEM((ZBLK, CH_F), jnp.float32),
        pltpu.VMEM((FROWS,), jnp.float32),
        pltpu.VMEM_SHARED((N_PAD, CH_F), jnp.float32),
        pltpu.VMEM_SHARED((N_PAD,), jnp.float32),
        pltpu.SemaphoreType.DMA,
    ],
)
def _sc_phase(table, gidx, sidx, acc_out, deg_out,
              gv, sv, rows, ones_v, zrow, zdeg,
              acc_sh, deg_sh, sem):
    c = lax.axis_index("c")
    s = lax.axis_index("s")
    wid = c * NS + s
    base = wid * PER_W

    zero16 = jnp.zeros((16,), jnp.float32)

    @pl.loop(0, CHUNK // 16)
    def _fill_ones(i):
        ones_v[pl.ds(i * 16, 16)] = jnp.ones((16,), jnp.float32)

    @pl.loop(0, ZBLK)
    def _fill_zrow(i):
        for j in range(CH_F // 16):
            zrow[i, pl.ds(j * 16, 16)] = zero16

    @pl.loop(0, FROWS // 16)
    def _fill_zdeg(i):
        zdeg[pl.ds(i * 16, 16)] = zero16

    z0 = s * FROWS
    for k in range(FROWS // ZBLK):
        pltpu.sync_copy(zrow, acc_sh.at[pl.ds(z0 + k * ZBLK, ZBLK)])
    pltpu.sync_copy(zdeg, deg_sh.at[pl.ds(z0, FROWS)])
    plsc.subcore_barrier()

    @pl.loop(0, N_CH)
    def _chunk(ci):
        off = base + ci * CHUNK
        pltpu.sync_copy(gidx.at[pl.ds(off, CHUNK)], gv)
        pltpu.sync_copy(sidx.at[pl.ds(off, CHUNK)], sv)
        pltpu.async_copy(table.at[gv], rows, sem).wait()
        pltpu.sync_copy(rows, acc_sh.at[sv], add=True)
        pltpu.sync_copy(ones_v, deg_sh.at[sv], add=True)

    plsc.subcore_barrier()

    pltpu.sync_copy(acc_sh.at[pl.ds(z0, FROWS)],
                    acc_out.at[c, pl.ds(z0, FROWS)])
    pltpu.sync_copy(deg_sh.at[pl.ds(z0, FROWS)],
                    deg_out.at[c, pl.ds(z0, FROWS)])


_RB = 1024


def _combine1_body(acc_ref, deg_ref, w_ref, out_ref):
    e = acc_ref[0] + acc_ref[1]
    d = (deg_ref[0] + deg_ref[1])[:, None]
    t = lax.dot_general(e, w_ref[...], (((1,), (1,)), ((), ())),
                        preferred_element_type=jnp.float32)
    out_ref[...] = jnp.where(d > 0, t / jnp.where(d > 0, d, 1.0), 0.0)


def _combine2_body(acc_ref, deg_ref, bias_ref, out_ref):
    e = acc_ref[0] + acc_ref[1]
    d = (deg_ref[0] + deg_ref[1])[:, None]
    out_ref[...] = jnp.where(d > 0, e / jnp.where(d > 0, d, 1.0), 0.0) \
        + bias_ref[...]


def _combine1(acc, deg, W):
    return pl.pallas_call(
        _combine1_body,
        grid=(N_PAD // _RB,),
        in_specs=[
            pl.BlockSpec((NC, _RB, CH_F), lambda i: (0, i, 0)),
            pl.BlockSpec((NC, _RB), lambda i: (0, i)),
            pl.BlockSpec((CH_F, CH_F), lambda i: (0, 0)),
        ],
        out_specs=pl.BlockSpec((_RB, CH_F), lambda i: (i, 0)),
        out_shape=jax.ShapeDtypeStruct((N_PAD, CH_F), jnp.float32),
    )(acc, deg, W)


def _combine2(acc, deg, bias2d):
    return pl.pallas_call(
        _combine2_body,
        grid=(N_PAD // _RB,),
        in_specs=[
            pl.BlockSpec((NC, _RB, CH_F), lambda i: (0, i, 0)),
            pl.BlockSpec((NC, _RB), lambda i: (0, i)),
            pl.BlockSpec((1, CH_F), lambda i: (0, 0)),
        ],
        out_specs=pl.BlockSpec((_RB, CH_F), lambda i: (i, 0)),
        out_shape=jax.ShapeDtypeStruct((N_PAD, CH_F), jnp.float32),
    )(acc, deg, bias2d)


def _pad_gather(idx):
    pad = jnp.zeros((INC_PAD - N_INC,), jnp.int32)
    return jnp.concatenate([idx, pad])


def _pad_scatter(idx):
    pad = N_ROWS + (jnp.arange(INC_PAD - N_INC, dtype=jnp.int32)
                    % (N_PAD - N_ROWS))
    return jnp.concatenate([idx, pad])


def kernel(x, hyperedge_index, W, bias):
    idx_node = hyperedge_index[0].astype(jnp.int32)
    idx_edge = hyperedge_index[1].astype(jnp.int32)
    node_g = _pad_gather(idx_node)
    node_s = _pad_scatter(idx_node)
    edge_g = _pad_gather(idx_edge)
    edge_s = _pad_scatter(idx_edge)

    acc_e, deg_e = _sc_phase(x, node_g, edge_s)
    t_edge = _combine1(acc_e, deg_e, W)
    acc_n, deg_n = _sc_phase(t_edge, edge_g, node_s)
    out = _combine2(acc_n, deg_n, bias.reshape(1, CH_F))
    return out[:N_ROWS]


# speedup vs baseline: 2.4677x; 2.4673x over previous
import functools

import jax
import jax.numpy as jnp
from jax import lax
from jax.experimental import pallas as pl
from jax.experimental.pallas import tpu as pltpu
from jax.experimental.pallas import tpu_sc as plsc

N_ROWS = 10000
N_PAD = 10240
N_INC = 320000
CH_F = 128
NC = 2
NS = 16
NW = NC * NS
CHUNK = 128
N_CH = 80
PER_W = N_CH * CHUNK
INC_PAD = NW * PER_W
FROWS = N_PAD // NS
ZBLK = 128

_mesh = plsc.VectorSubcoreMesh(core_axis_name="c", subcore_axis_name="s")


@functools.partial(
    pl.kernel,
    out_type=[
        jax.ShapeDtypeStruct((NC, N_PAD, CH_F), jnp.float32),
        jax.ShapeDtypeStruct((NC, N_PAD), jnp.float32),
    ],
    mesh=_mesh,
    scratch_types=[
        pltpu.VMEM((CHUNK,), jnp.int32),
        pltpu.VMEM((CHUNK,), jnp.int32),
        pltpu.VMEM((CHUNK, CH_F), jnp.float32),
        pltpu.VMEM((CHUNK,), jnp.float32),
        pltpu.VMEM((ZBLK, CH_F), jnp.float32),
        pltpu.VMEM((FROWS,), jnp.float32),
        pltpu.VMEM_SHARED((N_PAD, CH_F), jnp.float32),
        pltpu.VMEM_SHARED((N_PAD,), jnp.float32),
        pltpu.SemaphoreType.DMA,
    ],
)
def _sc_phase(table, gidx, sidx, acc_out, deg_out,
              gv, sv, rows, ones_v, zrow, zdeg,
              acc_sh, deg_sh, sem):
    c = lax.axis_index("c")
    s = lax.axis_index("s")
    wid = c * NS + s
    base = wid * PER_W

    zero16 = jnp.zeros((16,), jnp.float32)

    @pl.loop(0, CHUNK // 16)
    def _fill_ones(i):
        ones_v[pl.ds(i * 16, 16)] = jnp.ones((16,), jnp.float32)

    @pl.loop(0, ZBLK)
    def _fill_zrow(i):
        for j in range(CH_F // 16):
            zrow[i, pl.ds(j * 16, 16)] = zero16

    @pl.loop(0, FROWS // 16)
    def _fill_zdeg(i):
        zdeg[pl.ds(i * 16, 16)] = zero16

    z0 = s * FROWS
    for k in range(FROWS // ZBLK):
        pltpu.sync_copy(zrow, acc_sh.at[pl.ds(z0 + k * ZBLK, ZBLK)])
    pltpu.sync_copy(zdeg, deg_sh.at[pl.ds(z0, FROWS)])
    plsc.subcore_barrier()

    @pl.loop(0, N_CH)
    def _chunk(ci):
        off = base + ci * CHUNK
        pltpu.sync_copy(gidx.at[pl.ds(off, CHUNK)], gv)
        pltpu.sync_copy(sidx.at[pl.ds(off, CHUNK)], sv)
        pltpu.async_copy(table.at[gv], rows, sem).wait()
        pltpu.sync_copy(rows, acc_sh.at[sv], add=True)
        pltpu.sync_copy(ones_v, deg_sh.at[sv], add=True)

    plsc.subcore_barrier()

    pltpu.sync_copy(acc_sh.at[pl.ds(z0, FROWS)],
                    acc_out.at[c, pl.ds(z0, FROWS)])
    pltpu.sync_copy(deg_sh.at[pl.ds(z0, FROWS)],
                    deg_out.at[c, pl.ds(z0, FROWS)])


_RB = 1024


def _combine1_body(acc_ref, deg_ref, w_ref, out_ref):
    e = acc_ref[0] + acc_ref[1]
    d = (deg_ref[0] + deg_ref[1])[:, None]
    t = lax.dot_general(e, w_ref[...], (((1,), (1,)), ((), ())),
                        preferred_element_type=jnp.float32)
    out_ref[...] = jnp.where(d > 0, t / jnp.where(d > 0, d, 1.0), 0.0)


def _combine2_body(acc_ref, deg_ref, bias_ref, out_ref):
    e = acc_ref[0] + acc_ref[1]
    d = (deg_ref[0] + deg_ref[1])[:, None]
    out_ref[...] = jnp.where(d > 0, e / jnp.where(d > 0, d, 1.0), 0.0) \
        + bias_ref[...]


def _combine1(acc, deg, W):
    return pl.pallas_call(
        _combine1_body,
        grid=(N_PAD // _RB,),
        in_specs=[
            pl.BlockSpec((NC, _RB, CH_F), lambda i: (0, i, 0)),
            pl.BlockSpec((NC, _RB), lambda i: (0, i)),
            pl.BlockSpec((CH_F, CH_F), lambda i: (0, 0)),
        ],
        out_specs=pl.BlockSpec((_RB, CH_F), lambda i: (i, 0)),
        out_shape=jax.ShapeDtypeStruct((N_PAD, CH_F), jnp.float32),
    )(acc, deg, W)


def _combine2(acc, deg, bias2d):
    return pl.pallas_call(
        _combine2_body,
        grid=(N_PAD // _RB,),
        in_specs=[
            pl.BlockSpec((NC, _RB, CH_F), lambda i: (0, i, 0)),
            pl.BlockSpec((NC, _RB), lambda i: (0, i)),
            pl.BlockSpec((1, CH_F), lambda i: (0, 0)),
        ],
        out_specs=pl.BlockSpec((_RB, CH_F), lambda i: (i, 0)),
        out_shape=jax.ShapeDtypeStruct((N_PAD, CH_F), jnp.float32),
    )(acc, deg, bias2d)


def _pad_gather(idx):
    pad = (jnp.arange(INC_PAD - N_INC, dtype=jnp.int32) * 37) % N_ROWS
    return jnp.concatenate([idx, pad])


def _pad_scatter(idx):
    pad = N_ROWS + (jnp.arange(INC_PAD - N_INC, dtype=jnp.int32)
                    % (N_PAD - N_ROWS))
    return jnp.concatenate([idx, pad])


def kernel(x, hyperedge_index, W, bias):
    idx_node = hyperedge_index[0].astype(jnp.int32)
    idx_edge = hyperedge_index[1].astype(jnp.int32)
    node_g = _pad_gather(idx_node)
    node_s = _pad_scatter(idx_node)
    edge_g = _pad_gather(idx_edge)
    edge_s = _pad_scatter(idx_edge)

    acc_e, deg_e = _sc_phase(x, node_g, edge_s)
    t_edge = _combine1(acc_e, deg_e, W)
    acc_n, deg_n = _sc_phase(t_edge, edge_g, node_s)
    out = _combine2(acc_n, deg_n, bias.reshape(1, CH_F))
    return out[:N_ROWS]


# speedup vs baseline: 3.8196x; 1.5478x over previous
import functools

import jax
import jax.numpy as jnp
from jax import lax
from jax.experimental import pallas as pl
from jax.experimental.pallas import tpu as pltpu
from jax.experimental.pallas import tpu_sc as plsc

N_ROWS = 10000
N_PAD = 10240
N_INC = 320000
CH_F = 128
NC = 2
NS = 16
NW = NC * NS
CHUNK = 128
N_CH = 80
PER_W = N_CH * CHUNK
INC_PAD = NW * PER_W
FROWS = N_PAD // NS
ZBLK = 32

_mesh = plsc.VectorSubcoreMesh(core_axis_name="c", subcore_axis_name="s")


@functools.partial(
    pl.kernel,
    out_type=[
        jax.ShapeDtypeStruct((NC, N_PAD, CH_F), jnp.float32),
        jax.ShapeDtypeStruct((NC, N_PAD), jnp.float32),
    ],
    mesh=_mesh,
    scratch_types=[
        [pltpu.VMEM((CHUNK,), jnp.int32) for _ in range(2)],
        [pltpu.VMEM((CHUNK,), jnp.int32) for _ in range(2)],
        [pltpu.VMEM((CHUNK, CH_F), jnp.float32) for _ in range(2)],
        pltpu.VMEM((CHUNK,), jnp.float32),
        pltpu.VMEM((ZBLK, CH_F), jnp.float32),
        pltpu.VMEM((FROWS,), jnp.float32),
        pltpu.VMEM_SHARED((N_PAD, CH_F), jnp.float32),
        pltpu.VMEM_SHARED((N_PAD,), jnp.float32),
        pltpu.SemaphoreType.DMA,
    ],
)
def _sc_phase(table, gidx, sidx, acc_out, deg_out,
              gv, sv, rows, ones_v, zrow, zdeg,
              acc_sh, deg_sh, sem):
    c = lax.axis_index("c")
    s = lax.axis_index("s")
    wid = c * NS + s
    base = wid * PER_W

    zero16 = jnp.zeros((16,), jnp.float32)

    @pl.loop(0, CHUNK // 16)
    def _fill_ones(i):
        ones_v[pl.ds(i * 16, 16)] = jnp.ones((16,), jnp.float32)

    @pl.loop(0, ZBLK)
    def _fill_zrow(i):
        for j in range(CH_F // 16):
            zrow[i, pl.ds(j * 16, 16)] = zero16

    @pl.loop(0, FROWS // 16)
    def _fill_zdeg(i):
        zdeg[pl.ds(i * 16, 16)] = zero16

    z0 = s * FROWS
    for k in range(FROWS // ZBLK):
        pltpu.sync_copy(zrow, acc_sh.at[pl.ds(z0 + k * ZBLK, ZBLK)])
    pltpu.sync_copy(zdeg, deg_sh.at[pl.ds(z0, FROWS)])
    plsc.subcore_barrier()

    def scatter(b):
        pltpu.sync_copy(rows[b], acc_sh.at[sv[b]], add=True)
        pltpu.sync_copy(ones_v, deg_sh.at[sv[b]], add=True)

    def load_idx(ci, k):
        off = base + ci * CHUNK
        pltpu.sync_copy(gidx.at[pl.ds(off, CHUNK)], gv[k])
        pltpu.sync_copy(sidx.at[pl.ds(off, CHUNK)], sv[k])

    load_idx(0, 0)
    g0 = pltpu.async_copy(table.at[gv[0]], rows[0], sem)
    load_idx(1, 1)
    g0.wait()

    @pl.loop(0, (N_CH - 2) // 2)
    def _outer(o):
        for j in range(2):
            ci = o * 2 + j + 1
            b = (j + 1) % 2
            nb = j % 2
            g = pltpu.async_copy(table.at[gv[b]], rows[b], sem)
            scatter(nb)
            load_idx(ci + 1, nb)
            g.wait()

    g = pltpu.async_copy(table.at[gv[1]], rows[1], sem)
    scatter(0)
    g.wait()
    scatter(1)

    plsc.subcore_barrier()

    pltpu.sync_copy(acc_sh.at[pl.ds(z0, FROWS)],
                    acc_out.at[c, pl.ds(z0, FROWS)])
    pltpu.sync_copy(deg_sh.at[pl.ds(z0, FROWS)],
                    deg_out.at[c, pl.ds(z0, FROWS)])


_RB = 1024


def _combine1_body(acc_ref, deg_ref, w_ref, out_ref):
    e = acc_ref[0] + acc_ref[1]
    d = (deg_ref[0] + deg_ref[1])[:, None]
    t = lax.dot_general(e, w_ref[...], (((1,), (1,)), ((), ())),
                        preferred_element_type=jnp.float32)
    out_ref[...] = jnp.where(d > 0, t / jnp.where(d > 0, d, 1.0), 0.0)


def _combine2_body(acc_ref, deg_ref, bias_ref, out_ref):
    e = acc_ref[0] + acc_ref[1]
    d = (deg_ref[0] + deg_ref[1])[:, None]
    out_ref[...] = jnp.where(d > 0, e / jnp.where(d > 0, d, 1.0), 0.0) \
        + bias_ref[...]


def _combine1(acc, deg, W):
    return pl.pallas_call(
        _combine1_body,
        grid=(N_PAD // _RB,),
        in_specs=[
            pl.BlockSpec((NC, _RB, CH_F), lambda i: (0, i, 0)),
            pl.BlockSpec((NC, _RB), lambda i: (0, i)),
            pl.BlockSpec((CH_F, CH_F), lambda i: (0, 0)),
        ],
        out_specs=pl.BlockSpec((_RB, CH_F), lambda i: (i, 0)),
        out_shape=jax.ShapeDtypeStruct((N_PAD, CH_F), jnp.float32),
    )(acc, deg, W)


def _combine2(acc, deg, bias2d):
    return pl.pallas_call(
        _combine2_body,
        grid=(N_PAD // _RB,),
        in_specs=[
            pl.BlockSpec((NC, _RB, CH_F), lambda i: (0, i, 0)),
            pl.BlockSpec((NC, _RB), lambda i: (0, i)),
            pl.BlockSpec((1, CH_F), lambda i: (0, 0)),
        ],
        out_specs=pl.BlockSpec((_RB, CH_F), lambda i: (i, 0)),
        out_shape=jax.ShapeDtypeStruct((N_PAD, CH_F), jnp.float32),
    )(acc, deg, bias2d)


def _pad_gather(idx):
    pad = (jnp.arange(INC_PAD - N_INC, dtype=jnp.int32) * 37) % N_ROWS
    return jnp.concatenate([idx, pad])


def _pad_scatter(idx):
    pad = N_ROWS + (jnp.arange(INC_PAD - N_INC, dtype=jnp.int32)
                    % (N_PAD - N_ROWS))
    return jnp.concatenate([idx, pad])


def kernel(x, hyperedge_index, W, bias):
    idx_node = hyperedge_index[0].astype(jnp.int32)
    idx_edge = hyperedge_index[1].astype(jnp.int32)
    node_g = _pad_gather(idx_node)
    node_s = _pad_scatter(idx_node)
    edge_g = _pad_gather(idx_edge)
    edge_s = _pad_scatter(idx_edge)

    acc_e, deg_e = _sc_phase(x, node_g, edge_s)
    t_edge = _combine1(acc_e, deg_e, W)
    acc_n, deg_n = _sc_phase(t_edge, edge_g, node_s)
    out = _combine2(acc_n, deg_n, bias.reshape(1, CH_F))
    return out[:N_ROWS]
